# baseline (speedup 1.0000x reference)
import functools

import jax
import jax.numpy as jnp
from jax import lax
from jax.experimental import pallas as pl
from jax.experimental.pallas import tpu as pltpu
from jax.experimental.pallas import tpu_sc as plsc

B = 4
H = 224
W = 224
NPIX = B * H * W
NTILES = 32
PIX_PER_TILE = NPIX // NTILES
ROWS_PER_TILE = PIX_PER_TILE // W
CHUNK = W // 2
NCHUNK = PIX_PER_TILE // CHUNK
VEC = 16


def _warp_body(t0, t1, t2, dx_hbm, dy_hbm, out_hbm, dx_v, dy_v, idx_v, grows, sem):
    wid = lax.axis_index("s") * 2 + lax.axis_index("c")
    bb = wid >> 3
    i0 = (wid & 7) * ROWS_PER_TILE
    base = wid * PIX_PER_TILE
    pltpu.sync_copy(dx_hbm.at[pl.ds(base, PIX_PER_TILE)], dx_v)
    pltpu.sync_copy(dy_hbm.at[pl.ds(base, PIX_PER_TILE)], dy_v)

    lane = lax.iota(jnp.int32, VEC)
    bf = bb.astype(jnp.float32)
    g0 = jnp.minimum(lane, 3) * (H * W)

    def row_body(r, carry):
        roff = r * W
        fi = (i0 + r).astype(jnp.float32)
        for v in range(W // VEC):
            off = roff + v * VEC
            fy = bf + dy_v[pl.ds(off, VEC)]
            fx = fi + dx_v[pl.ds(off, VEC)]
            yy = jnp.minimum(jnp.mod(fy, 224.0).astype(jnp.int32), H - 1)
            xx = jnp.minimum(jnp.mod(fx, 224.0).astype(jnp.int32), W - 1)
            goff = g0 if v == 0 else 3 * (H * W)
            idx_v[pl.ds(off, VEC)] = goff + yy * W + xx
        return carry

    lax.fori_loop(0, ROWS_PER_TILE, row_body, 0)

    tables = (t0, t1, t2)
    handles = [
        pltpu.async_copy(
            tables[ch].at[idx_v],
            grows.at[pl.ds(ch * PIX_PER_TILE, PIX_PER_TILE)],
            sem,
        )
        for ch in range(3)
    ]
    for h in handles:
        h.wait()

    for ch in range(3):
        pltpu.sync_copy(
            grows.at[pl.ds(ch * PIX_PER_TILE, PIX_PER_TILE)],
            out_hbm.at[pl.ds(((bb * 3 + ch) * H + i0) * W, PIX_PER_TILE)],
        )


_warp = functools.partial(
    pl.kernel,
    out_type=jax.ShapeDtypeStruct((B * 3 * H * W,), jnp.float32),
    mesh=plsc.VectorSubcoreMesh(core_axis_name="c", subcore_axis_name="s"),
    scratch_types=[
        pltpu.VMEM((PIX_PER_TILE,), jnp.float32),
        pltpu.VMEM((PIX_PER_TILE,), jnp.float32),
        pltpu.VMEM((PIX_PER_TILE,), jnp.int32),
        pltpu.VMEM((3 * PIX_PER_TILE,), jnp.float32),
        pltpu.SemaphoreType.DMA,
    ],
    compiler_params=pltpu.CompilerParams(use_tc_tiling_on_sc=False),
)(_warp_body)


def kernel(x):
    xp = jnp.transpose(x, (3, 0, 1, 2))
    tabs = [xp[ch].reshape(NPIX) for ch in range(3)]
    dxf = xp[3].reshape(NPIX)
    dyf = xp[4].reshape(NPIX)
    res = _warp(*tabs, dxf, dyf)
    return jnp.transpose(res.reshape(B, 3, H, W), (0, 2, 3, 1))


# speedup vs baseline: 1.7665x; 1.7665x over previous
import functools

import jax
import jax.numpy as jnp
from jax import lax
from jax.experimental import pallas as pl
from jax.experimental.pallas import tpu as pltpu
from jax.experimental.pallas import tpu_sc as plsc

B = 4
H = 224
W = 224
NPIX = B * H * W
NTILES = 32
PIX_PER_TILE = NPIX // NTILES
ROWS_PER_TILE = PIX_PER_TILE // W
CHUNK = W // 2
NCHUNK = PIX_PER_TILE // CHUNK
GCHUNK = 448
VEC = 16


def _warp_body(t0, t1, t2, dx_hbm, dy_hbm, out_hbm, dx_v, dy_v, idx_v, grows, sem):
    wid = lax.axis_index("s") * 2 + lax.axis_index("c")
    bb = wid >> 3
    i0 = (wid & 7) * ROWS_PER_TILE
    base = wid * PIX_PER_TILE
    pltpu.sync_copy(dx_hbm.at[pl.ds(base, PIX_PER_TILE)], dx_v)
    pltpu.sync_copy(dy_hbm.at[pl.ds(base, PIX_PER_TILE)], dy_v)

    lane = lax.iota(jnp.int32, VEC)
    bf = bb.astype(jnp.float32)
    g0 = jnp.minimum(lane, 3) * (H * W)

    def row_body(r, carry):
        roff = r * W
        fi = (i0 + r).astype(jnp.float32)
        for v in range(W // VEC):
            off = roff + v * VEC
            fy = bf + dy_v[pl.ds(off, VEC)]
            fx = fi + dx_v[pl.ds(off, VEC)]
            yy = jnp.minimum(jnp.mod(fy, 224.0).astype(jnp.int32), H - 1)
            xx = jnp.minimum(jnp.mod(fx, 224.0).astype(jnp.int32), W - 1)
            goff = g0 if v == 0 else 3 * (H * W)
            idx_v[pl.ds(off, VEC)] = goff + yy * W + xx
        return carry

    lax.fori_loop(0, ROWS_PER_TILE, row_body, 0)

    tables = (t0, t1, t2)
    handles = [
        pltpu.async_copy(
            tables[ch].at[idx_v.at[pl.ds(c * GCHUNK, GCHUNK)]],
            grows.at[pl.ds(ch * PIX_PER_TILE + c * GCHUNK, GCHUNK)],
            sem,
        )
        for c in range(PIX_PER_TILE // GCHUNK)
        for ch in range(3)
    ]
    for h in handles:
        h.wait()

    for ch in range(3):
        pltpu.sync_copy(
            grows.at[pl.ds(ch * PIX_PER_TILE, PIX_PER_TILE)],
            out_hbm.at[pl.ds(((bb * 3 + ch) * H + i0) * W, PIX_PER_TILE)],
        )


_warp = functools.partial(
    pl.kernel,
    out_type=jax.ShapeDtypeStruct((B * 3 * H * W,), jnp.float32),
    mesh=plsc.VectorSubcoreMesh(core_axis_name="c", subcore_axis_name="s"),
    scratch_types=[
        pltpu.VMEM((PIX_PER_TILE,), jnp.float32),
        pltpu.VMEM((PIX_PER_TILE,), jnp.float32),
        pltpu.VMEM((PIX_PER_TILE,), jnp.int32),
        pltpu.VMEM((3 * PIX_PER_TILE,), jnp.float32),
        pltpu.SemaphoreType.DMA,
    ],
    compiler_params=pltpu.CompilerParams(use_tc_tiling_on_sc=False),
)(_warp_body)


def kernel(x):
    xp = jnp.transpose(x, (3, 0, 1, 2))
    tabs = [xp[ch].reshape(NPIX) for ch in range(3)]
    dxf = xp[3].reshape(NPIX)
    dyf = xp[4].reshape(NPIX)
    res = _warp(*tabs, dxf, dyf)
    return jnp.transpose(res.reshape(B, 3, H, W), (0, 2, 3, 1))


# speedup vs baseline: 1.7764x; 1.0056x over previous
import functools

import jax
import jax.numpy as jnp
from jax import lax
from jax.experimental import pallas as pl
from jax.experimental.pallas import tpu as pltpu
from jax.experimental.pallas import tpu_sc as plsc

B = 4
H = 224
W = 224
NPIX = B * H * W
NTILES = 32
PIX_PER_TILE = NPIX // NTILES
ROWS_PER_TILE = PIX_PER_TILE // W
CHUNK = W // 2
NCHUNK = PIX_PER_TILE // CHUNK
GCHUNK = 224
VEC = 16


def _warp_body(t0, t1, t2, dx_hbm, dy_hbm, out_hbm, dx_v, dy_v, idx_v, grows, sem):
    wid = lax.axis_index("s") * 2 + lax.axis_index("c")
    bb = wid >> 3
    i0 = (wid & 7) * ROWS_PER_TILE
    base = wid * PIX_PER_TILE
    pltpu.sync_copy(dx_hbm.at[pl.ds(base, PIX_PER_TILE)], dx_v)
    pltpu.sync_copy(dy_hbm.at[pl.ds(base, PIX_PER_TILE)], dy_v)

    lane = lax.iota(jnp.int32, VEC)
    bf = bb.astype(jnp.float32)
    g0 = jnp.minimum(lane, 3) * (H * W)

    def row_body(r, carry):
        roff = r * W
        fi = (i0 + r).astype(jnp.float32)
        for v in range(W // VEC):
            off = roff + v * VEC
            fy = bf + dy_v[pl.ds(off, VEC)]
            fx = fi + dx_v[pl.ds(off, VEC)]
            yy = jnp.minimum(jnp.mod(fy, 224.0).astype(jnp.int32), H - 1)
            xx = jnp.minimum(jnp.mod(fx, 224.0).astype(jnp.int32), W - 1)
            goff = g0 if v == 0 else 3 * (H * W)
            idx_v[pl.ds(off, VEC)] = goff + yy * W + xx
        return carry

    lax.fori_loop(0, ROWS_PER_TILE, row_body, 0)

    tables = (t0, t1, t2)
    handles = [
        pltpu.async_copy(
            tables[ch].at[idx_v.at[pl.ds(c * GCHUNK, GCHUNK)]],
            grows.at[pl.ds(ch * PIX_PER_TILE + c * GCHUNK, GCHUNK)],
            sem,
        )
        for c in range(PIX_PER_TILE // GCHUNK)
        for ch in range(3)
    ]
    for h in handles:
        h.wait()

    for ch in range(3):
        pltpu.sync_copy(
            grows.at[pl.ds(ch * PIX_PER_TILE, PIX_PER_TILE)],
            out_hbm.at[pl.ds(((bb * 3 + ch) * H + i0) * W, PIX_PER_TILE)],
        )


_warp = functools.partial(
    pl.kernel,
    out_type=jax.ShapeDtypeStruct((B * 3 * H * W,), jnp.float32),
    mesh=plsc.VectorSubcoreMesh(core_axis_name="c", subcore_axis_name="s"),
    scratch_types=[
        pltpu.VMEM((PIX_PER_TILE,), jnp.float32),
        pltpu.VMEM((PIX_PER_TILE,), jnp.float32),
        pltpu.VMEM((PIX_PER_TILE,), jnp.int32),
        pltpu.VMEM((3 * PIX_PER_TILE,), jnp.float32),
        pltpu.SemaphoreType.DMA,
    ],
    compiler_params=pltpu.CompilerParams(use_tc_tiling_on_sc=False),
)(_warp_body)


def kernel(x):
    xp = jnp.transpose(x, (3, 0, 1, 2))
    tabs = [xp[ch].reshape(NPIX) for ch in range(3)]
    dxf = xp[3].reshape(NPIX)
    dyf = xp[4].reshape(NPIX)
    res = _warp(*tabs, dxf, dyf)
    return jnp.transpose(res.reshape(B, 3, H, W), (0, 2, 3, 1))
